# baseline (speedup 1.0000x reference)
import jax
import jax.numpy as jnp
from jax.experimental import pallas as pl

B, A, D, KM1, K_TOP = 64, 4096, 256, 64, 64
A_TILE = 2048


def _score_body(x_ref, w_ref, b_ref, o_ref):
    x = x_ref[...].reshape(A_TILE, D)
    logits_t = jax.lax.dot_general(
        w_ref[...], x, (((0,), (1,)), ((), ())),
        preferred_element_type=jnp.float32)
    s = jax.nn.sigmoid(logits_t + b_ref[...])
    o_ref[...] = (s.sum(axis=0) * (1.0 / KM1) - 0.5).reshape(1, 1, A_TILE)


def _select_body(s_ref, o_ref):
    st = s_ref[...].reshape(B, A).T
    at = jnp.abs(st)
    bits = jax.lax.bitcast_convert_type(at, jnp.int32)
    iota = jax.lax.broadcasted_iota(jnp.int32, (A, B), 0)

    def vstep(_, c):
        lo, hi = c
        mid = lo + ((hi - lo) >> 1)
        cnt = jnp.sum((bits >= mid).astype(jnp.int32), axis=0, keepdims=True)
        ge = cnt >= K_TOP
        return jnp.where(ge, mid, lo), jnp.where(ge, hi, mid)

    t, _ = jax.lax.fori_loop(
        0, 31, vstep,
        (jnp.zeros((1, B), jnp.int32), jnp.full((1, B), 0x7F800000, jnp.int32)))

    gt = bits > t
    eq = bits == t
    n_gt = jnp.sum(gt.astype(jnp.int32), axis=0, keepdims=True)

    def istep(_, c):
        lo, hi = c
        mid = lo + ((hi - lo + 1) >> 1)
        cnt = n_gt + jnp.sum((eq & (iota <= mid)).astype(jnp.int32),
                             axis=0, keepdims=True)
        ge = cnt >= K_TOP
        return jnp.where(ge, lo, mid), jnp.where(ge, mid, hi)

    _, i_thr = jax.lax.fori_loop(
        0, 12, istep,
        (jnp.full((1, B), -1, jnp.int32), jnp.full((1, B), A - 1, jnp.int32)))

    mask = gt | (eq & (iota <= i_thr))
    sel = jnp.where(mask, st, 0.0)
    z = jnp.sum(jnp.abs(sel), axis=0, keepdims=True)
    o_ref[...] = (sel / (z + 1e-8)).T


@jax.jit
def kernel(signal_features, W, b):
    scores = pl.pallas_call(
        _score_body,
        grid=(B, A // A_TILE),
        in_specs=[
            pl.BlockSpec((1, A_TILE, D), lambda i, j: (i, j, 0)),
            pl.BlockSpec((D, KM1), lambda i, j: (0, 0)),
            pl.BlockSpec((KM1, 1), lambda i, j: (0, 0)),
        ],
        out_specs=pl.BlockSpec((1, 1, A_TILE), lambda i, j: (i, 0, j)),
        out_shape=jax.ShapeDtypeStruct((B, 1, A), jnp.float32),
    )(signal_features, W, b.reshape(KM1, 1))

    action = pl.pallas_call(
        _select_body,
        out_shape=jax.ShapeDtypeStruct((B, A), jnp.float32),
    )(scores)
    return action


# speedup vs baseline: 1.5438x; 1.5438x over previous
import jax
import jax.numpy as jnp
from jax.experimental import pallas as pl

B, A, D, KM1, K_TOP = 64, 4096, 256, 64, 64
B_TILE = 2


def _score_body(x_ref, w_ref, b_ref, o_ref):
    x = x_ref[...].reshape(B_TILE * A, D)
    logits_t = jax.lax.dot_general(
        w_ref[...], x, (((0,), (1,)), ((), ())),
        preferred_element_type=jnp.float32)
    s = jax.nn.sigmoid(logits_t + b_ref[...])
    o_ref[...] = (s.sum(axis=0) * (1.0 / KM1) - 0.5).reshape(B_TILE, 1, A)


def _select_body(s_ref, o_ref):
    st = s_ref[...].reshape(B, A).T
    at = jnp.abs(st)
    bits = jax.lax.bitcast_convert_type(at, jnp.int32)
    iota = jax.lax.broadcasted_iota(jnp.int32, (A, B), 0)

    def vstep(_, c):
        lo, hi = c
        mid = lo + ((hi - lo) >> 1)
        cnt = jnp.sum((bits >= mid).astype(jnp.int32), axis=0, keepdims=True)
        ge = cnt >= K_TOP
        return jnp.where(ge, mid, lo), jnp.where(ge, hi, mid)

    t, _ = jax.lax.fori_loop(
        0, 31, vstep,
        (jnp.zeros((1, B), jnp.int32), jnp.full((1, B), 0x7F800000, jnp.int32)))

    gt = bits > t
    eq = bits == t
    n_gt = jnp.sum(gt.astype(jnp.int32), axis=0, keepdims=True)

    def istep(_, c):
        lo, hi = c
        mid = lo + ((hi - lo + 1) >> 1)
        cnt = n_gt + jnp.sum((eq & (iota <= mid)).astype(jnp.int32),
                             axis=0, keepdims=True)
        ge = cnt >= K_TOP
        return jnp.where(ge, lo, mid), jnp.where(ge, mid, hi)

    _, i_thr = jax.lax.fori_loop(
        0, 12, istep,
        (jnp.full((1, B), -1, jnp.int32), jnp.full((1, B), A - 1, jnp.int32)))

    mask = gt | (eq & (iota <= i_thr))
    sel = jnp.where(mask, st, 0.0)
    z = jnp.sum(jnp.abs(sel), axis=0, keepdims=True)
    o_ref[...] = (sel / (z + 1e-8)).T


@jax.jit
def kernel(signal_features, W, b):
    scores = pl.pallas_call(
        _score_body,
        grid=(B // B_TILE,),
        in_specs=[
            pl.BlockSpec((B_TILE, A, D), lambda i: (i, 0, 0)),
            pl.BlockSpec((D, KM1), lambda i: (0, 0)),
            pl.BlockSpec((KM1, 1), lambda i: (0, 0)),
        ],
        out_specs=pl.BlockSpec((B_TILE, 1, A), lambda i: (i, 0, 0)),
        out_shape=jax.ShapeDtypeStruct((B, 1, A), jnp.float32),
    )(signal_features, W, b.reshape(KM1, 1))

    action = pl.pallas_call(
        _select_body,
        out_shape=jax.ShapeDtypeStruct((B, A), jnp.float32),
    )(scores)
    return action


# speedup vs baseline: 1.5707x; 1.0174x over previous
import jax
import jax.numpy as jnp
from jax.experimental import pallas as pl

B, A, D, KM1, K_TOP = 64, 4096, 256, 64, 64
B_TILE = 4


def _score_body(x_ref, w_ref, b_ref, o_ref):
    x = x_ref[...].reshape(B_TILE * A, D)
    logits_t = jax.lax.dot_general(
        w_ref[...], x, (((0,), (1,)), ((), ())),
        preferred_element_type=jnp.float32)
    s = jax.nn.sigmoid(logits_t + b_ref[...])
    o_ref[...] = (s.sum(axis=0) * (1.0 / KM1) - 0.5).reshape(B_TILE, 1, A)


def _select_body(s_ref, o_ref):
    st = s_ref[...].reshape(B, A).T
    at = jnp.abs(st)
    bits = jax.lax.bitcast_convert_type(at, jnp.int32)
    iota = jax.lax.broadcasted_iota(jnp.int32, (A, B), 0)

    def vstep(_, c):
        lo, hi = c
        mid = lo + ((hi - lo) >> 1)
        cnt = jnp.sum((bits >= mid).astype(jnp.int32), axis=0, keepdims=True)
        ge = cnt >= K_TOP
        return jnp.where(ge, mid, lo), jnp.where(ge, hi, mid)

    t, _ = jax.lax.fori_loop(
        0, 31, vstep,
        (jnp.zeros((1, B), jnp.int32), jnp.full((1, B), 0x7F800000, jnp.int32)))

    gt = bits > t
    eq = bits == t
    n_gt = jnp.sum(gt.astype(jnp.int32), axis=0, keepdims=True)

    def istep(_, c):
        lo, hi = c
        mid = lo + ((hi - lo + 1) >> 1)
        cnt = n_gt + jnp.sum((eq & (iota <= mid)).astype(jnp.int32),
                             axis=0, keepdims=True)
        ge = cnt >= K_TOP
        return jnp.where(ge, lo, mid), jnp.where(ge, mid, hi)

    _, i_thr = jax.lax.fori_loop(
        0, 12, istep,
        (jnp.full((1, B), -1, jnp.int32), jnp.full((1, B), A - 1, jnp.int32)))

    mask = gt | (eq & (iota <= i_thr))
    sel = jnp.where(mask, st, 0.0)
    z = jnp.sum(jnp.abs(sel), axis=0, keepdims=True)
    o_ref[...] = (sel / (z + 1e-8)).T


@jax.jit
def kernel(signal_features, W, b):
    scores = pl.pallas_call(
        _score_body,
        grid=(B // B_TILE,),
        in_specs=[
            pl.BlockSpec((B_TILE, A, D), lambda i: (i, 0, 0)),
            pl.BlockSpec((D, KM1), lambda i: (0, 0)),
            pl.BlockSpec((KM1, 1), lambda i: (0, 0)),
        ],
        out_specs=pl.BlockSpec((B_TILE, 1, A), lambda i: (i, 0, 0)),
        out_shape=jax.ShapeDtypeStruct((B, 1, A), jnp.float32),
    )(signal_features, W, b.reshape(KM1, 1))

    action = pl.pallas_call(
        _select_body,
        out_shape=jax.ShapeDtypeStruct((B, A), jnp.float32),
    )(scores)
    return action
